# baseline (speedup 1.0000x reference)
import functools

import jax
import jax.numpy as jnp
from jax import lax
from jax.experimental import pallas as pl
from jax.experimental.pallas import tpu as pltpu
from jax.experimental.pallas import tpu_sc as plsc

BATCH = 16384
VOCAB = 1000000
DEPTH = 64
NC = 2
NS = 16
NW = NC * NS
BPW = BATCH // NW

_mesh = plsc.VectorSubcoreMesh(core_axis_name="c", subcore_axis_name="s")


@functools.partial(
    pl.kernel,
    mesh=_mesh,
    out_type=jax.ShapeDtypeStruct((BATCH, DEPTH), jnp.float32),
    scratch_types=[
        pltpu.VMEM((BPW,), jnp.int32),
        pltpu.SemaphoreType.DMA,
    ],
)
def _gather_kernel(idx_hbm, table_hbm, out_hbm, idx_vm, sem):
    wid = lax.axis_index("s") * NC + lax.axis_index("c")
    pltpu.sync_copy(idx_hbm.at[wid], idx_vm)

    def body(g, carry):
        base = g * 16
        v = idx_vm[pl.ds(base, 16)]
        for l in range(16):
            pltpu.async_copy(
                table_hbm.at[v[l]],
                out_hbm.at[wid * BPW + base + l],
                sem)
        return carry

    lax.fori_loop(0, BPW // 16, body, 0)
    pltpu.make_async_copy(
        table_hbm.at[pl.ds(0, BPW)],
        out_hbm.at[pl.ds(wid * BPW, BPW)],
        sem).wait()


def kernel(x, embedding):
    idx = jnp.reshape(x, (NW, BPW))
    return _gather_kernel(idx, embedding)


# speedup vs baseline: 1.6705x; 1.6705x over previous
import functools

import jax
import jax.numpy as jnp
from jax import lax
from jax.experimental import pallas as pl
from jax.experimental.pallas import tpu as pltpu
from jax.experimental.pallas import tpu_sc as plsc

BATCH = 16384
VOCAB = 1000000
DEPTH = 64
NC = 2
NS = 16
NW = NC * NS
BPW = BATCH // NW

_mesh = plsc.VectorSubcoreMesh(core_axis_name="c", subcore_axis_name="s")


@functools.partial(
    pl.kernel,
    mesh=_mesh,
    out_type=jax.ShapeDtypeStruct((BATCH, DEPTH), jnp.float32),
    scratch_types=[
        pltpu.VMEM((BPW,), jnp.int32),
        pltpu.VMEM((BPW, DEPTH), jnp.float32),
        pltpu.SemaphoreType.DMA,
        pltpu.SemaphoreType.DMA,
        pltpu.SemaphoreType.DMA,
        pltpu.SemaphoreType.DMA,
        pltpu.SemaphoreType.DMA,
        pltpu.SemaphoreType.DMA,
        pltpu.SemaphoreType.DMA,
        pltpu.SemaphoreType.DMA,
    ],
)
def _gather_kernel(idx_hbm, table_hbm, out_hbm, idx_vm, stage_v, *sems):
    wid = lax.axis_index("s") * NC + lax.axis_index("c")
    pltpu.sync_copy(idx_hbm.at[wid], idx_vm)

    def body(g, carry):
        base = g * 16
        v = idx_vm[pl.ds(base, 16)]
        for l in range(16):
            pltpu.async_copy(
                table_hbm.at[v[l]], stage_v.at[base + l], sems[l % 8])
        return carry

    lax.fori_loop(0, BPW // 16, body, 0)
    for q in range(8):
        pltpu.make_async_copy(
            table_hbm.at[pl.ds(0, BPW // 8)],
            stage_v.at[pl.ds(q * (BPW // 8), BPW // 8)],
            sems[q]).wait()
    pltpu.sync_copy(stage_v, out_hbm.at[pl.ds(wid * BPW, BPW)])


def kernel(x, embedding):
    idx = jnp.reshape(x, (NW, BPW))
    return _gather_kernel(idx, embedding)
